# baseline (speedup 1.0000x reference)
import functools

import jax
import jax.numpy as jnp
from jax import lax
from jax.experimental import pallas as pl
from jax.experimental.pallas import tpu as pltpu
from jax.experimental.pallas import tpu_sc as plsc

N_NODES = 100000
DIM = 16
N_TICKS = 20
BATCH = 16384

NALN = 99968
NTAIL = N_NODES - NALN
H0 = 49920
S = NALN - H0

NC = 2
NS = 16
PER_T = BATCH // NS
DPC = DIM // NC
NQ = PER_T // 128
NG = PER_T // 16

STEP = 1.0 / (N_TICKS - 1)


def _body(src_hbm, dst_hbm, t_hbm, zt_hbm, ztail_hbm, out_hbm,
          sidx, didx, tv, dtv, omdv,
          a_s0, a_d0, a_s1, a_d1, a_sn0, a_dn0, a_sn1, a_dn1,
          tbs, tbd, f1s, f1d, fts, ftd,
          bsc0, bsn0, bdc0, bdn0, bsc1, bsn1, bdc1, bdn1,
          accv, tailv, plane, sem_p, sem_g):
    c = lax.axis_index("c")
    s = lax.axis_index("s")
    base = s * PER_T

    pltpu.sync_copy(src_hbm.at[pl.ds(base, PER_T)], sidx)
    pltpu.sync_copy(dst_hbm.at[pl.ds(base, PER_T)], didx)
    pltpu.sync_copy(t_hbm.at[pl.ds(base, PER_T)], tv)
    pltpu.sync_copy(ztail_hbm, tailv)

    iota16 = lax.iota(jnp.int32, 16)
    one = jnp.int32(1)
    zero = jnp.int32(0)

    def prep(i, _):
        off = i * 16
        tvec = tv[pl.ds(off, 16)]
        ti = (tvec / STEP).astype(jnp.int32)
        ti = jnp.minimum(ti, N_TICKS - 2)
        dt = lax.rem(tvec, STEP) / STEP
        dtv[pl.ds(off, 16)] = dt
        omdv[pl.ds(off, 16)] = jnp.float32(1.0) - dt
        sv = sidx[pl.ds(off, 16)]
        dv = didx[pl.ds(off, 16)]
        tiS = ti * S
        v_s0 = tiS + jnp.minimum(sv, H0 - 1)
        v_d0 = tiS + jnp.minimum(dv, H0 - 1)
        v_s1 = tiS + jnp.clip(sv - H0, 0, S - 1)
        v_d1 = tiS + jnp.clip(dv - H0, 0, S - 1)
        a_s0[pl.ds(off, 16)] = v_s0
        a_d0[pl.ds(off, 16)] = v_d0
        a_s1[pl.ds(off, 16)] = v_s1
        a_d1[pl.ds(off, 16)] = v_d1
        a_sn0[pl.ds(off, 16)] = v_s0 + S
        a_dn0[pl.ds(off, 16)] = v_d0 + S
        a_sn1[pl.ds(off, 16)] = v_s1 + S
        a_dn1[pl.ds(off, 16)] = v_d1 + S
        tbs[pl.ds(off, 16)] = jnp.maximum(sv - NALN, 0) * (DIM * N_TICKS) + ti
        tbd[pl.ds(off, 16)] = jnp.maximum(dv - NALN, 0) * (DIM * N_TICKS) + ti
        f1s[pl.ds(off, 16)] = jnp.where(sv >= H0, one, zero)
        f1d[pl.ds(off, 16)] = jnp.where(dv >= H0, one, zero)
        fts[pl.ds(off, 16)] = jnp.where(sv >= NALN, one, zero)
        ftd[pl.ds(off, 16)] = jnp.where(dv >= NALN, one, zero)
        accv[pl.ds(off, 16)] = jnp.zeros((16,), jnp.float32)
        return 0

    lax.fori_loop(0, NG, prep, 0)

    def load_half(d, node0, size):
        def issue_t(t, _):
            pltpu.async_copy(
                zt_hbm.at[t, d, pl.ds(node0, size)],
                plane.at[pl.ds(t * S, size)], sem_p)
            return 0

        n_mine = jnp.where(s < N_TICKS - NS, 2, 1)

        issue_t(s, 0)

        @pl.when(s < N_TICKS - NS)
        def _second():
            issue_t(s + NS, 0)

        def drain(i, _):
            pltpu.make_async_copy(
                zt_hbm.at[0, 0, pl.ds(node0, size)],
                plane.at[pl.ds(0, size)], sem_p).wait()
            return 0

        lax.fori_loop(0, n_mine, drain, 0)

    def gather_pass(a_s, a_sn, a_d, a_dn, bsc, bsn, bdc, bdn):
        pltpu.async_copy(plane.at[a_s], bsc, sem_g)
        pltpu.async_copy(plane.at[a_sn], bsn, sem_g)
        pltpu.async_copy(plane.at[a_d], bdc, sem_g)
        pltpu.async_copy(plane.at[a_dn], bdn, sem_g)
        pltpu.make_async_copy(plane.at[a_s], bsc, sem_g).wait()
        pltpu.make_async_copy(plane.at[a_sn], bsn, sem_g).wait()
        pltpu.make_async_copy(plane.at[a_d], bdc, sem_g).wait()
        pltpu.make_async_copy(plane.at[a_dn], bdn, sem_g).wait()

    def dim_step(dl, _):
        d = c * DPC + dl

        @pl.when(s == 0)
        def _l0():
            load_half(d, 0, H0)

        plsc.subcore_barrier()
        gather_pass(a_s0, a_sn0, a_d0, a_dn0, bsc0, bsn0, bdc0, bdn0)
        plsc.subcore_barrier()

        @pl.when(s == 0)
        def _l1():
            load_half(d, H0, S)

        plsc.subcore_barrier()
        gather_pass(a_s1, a_sn1, a_d1, a_dn1, bsc1, bsn1, bdc1, bdn1)

        def grp(i, _):
            off = i * 16
            dt = dtv[pl.ds(off, 16)]
            omd = omdv[pl.ds(off, 16)]
            h1s = f1s[pl.ds(off, 16)] > 0
            h1d = f1d[pl.ds(off, 16)] > 0
            tls = fts[pl.ds(off, 16)] > 0
            tld = ftd[pl.ds(off, 16)] > 0
            its = tbs[pl.ds(off, 16)] + d * N_TICKS
            itd = tbd[pl.ds(off, 16)] + d * N_TICKS
            s_cur = jnp.where(h1s, bsc1[pl.ds(off, 16)], bsc0[pl.ds(off, 16)])
            s_nxt = jnp.where(h1s, bsn1[pl.ds(off, 16)], bsn0[pl.ds(off, 16)])
            d_cur = jnp.where(h1d, bdc1[pl.ds(off, 16)], bdc0[pl.ds(off, 16)])
            d_nxt = jnp.where(h1d, bdn1[pl.ds(off, 16)], bdn0[pl.ds(off, 16)])
            s_cur = jnp.where(tls, plsc.load_gather(tailv, [its]), s_cur)
            s_nxt = jnp.where(tls, plsc.load_gather(tailv, [its + 1]), s_nxt)
            d_cur = jnp.where(tld, plsc.load_gather(tailv, [itd]), d_cur)
            d_nxt = jnp.where(tld, plsc.load_gather(tailv, [itd + 1]), d_nxt)
            zs = omd * s_cur + dt * s_nxt
            zd = omd * d_cur + dt * d_nxt
            diff = zs - zd
            accv[pl.ds(off, 16)] = accv[pl.ds(off, 16)] + diff * diff
            return 0

        lax.fori_loop(0, NG, grp, 0)

        plsc.subcore_barrier()
        return 0

    lax.fori_loop(0, DPC, dim_step, 0)

    pltpu.sync_copy(accv, out_hbm.at[c, pl.ds(base, PER_T)])


def kernel(src, dst, t, z, bias):
    zt = jnp.transpose(z, (2, 1, 0))
    ztail = z[NALN:].reshape(NTAIL * DIM * N_TICKS)
    src32 = src.astype(jnp.int32)
    dst32 = dst.astype(jnp.int32)

    mesh = plsc.VectorSubcoreMesh(core_axis_name="c", subcore_axis_name="s")
    k = functools.partial(
        pl.kernel,
        mesh=mesh,
        compiler_params=pltpu.CompilerParams(needs_layout_passes=False),
        out_type=jax.ShapeDtypeStruct((NC, BATCH), jnp.float32),
        scratch_types=[
            pltpu.VMEM((PER_T,), jnp.int32),
            pltpu.VMEM((PER_T,), jnp.int32),
            pltpu.VMEM((PER_T,), jnp.float32),
            pltpu.VMEM((PER_T,), jnp.float32),
            pltpu.VMEM((PER_T,), jnp.float32),
            pltpu.VMEM((PER_T,), jnp.int32),
            pltpu.VMEM((PER_T,), jnp.int32),
            pltpu.VMEM((PER_T,), jnp.int32),
            pltpu.VMEM((PER_T,), jnp.int32),
            pltpu.VMEM((PER_T,), jnp.int32),
            pltpu.VMEM((PER_T,), jnp.int32),
            pltpu.VMEM((PER_T,), jnp.int32),
            pltpu.VMEM((PER_T,), jnp.int32),
            pltpu.VMEM((PER_T,), jnp.int32),
            pltpu.VMEM((PER_T,), jnp.int32),
            pltpu.VMEM((PER_T,), jnp.int32),
            pltpu.VMEM((PER_T,), jnp.int32),
            pltpu.VMEM((PER_T,), jnp.int32),
            pltpu.VMEM((PER_T,), jnp.int32),
            pltpu.VMEM((PER_T,), jnp.float32),
            pltpu.VMEM((PER_T,), jnp.float32),
            pltpu.VMEM((PER_T,), jnp.float32),
            pltpu.VMEM((PER_T,), jnp.float32),
            pltpu.VMEM((PER_T,), jnp.float32),
            pltpu.VMEM((PER_T,), jnp.float32),
            pltpu.VMEM((PER_T,), jnp.float32),
            pltpu.VMEM((PER_T,), jnp.float32),
            pltpu.VMEM((PER_T,), jnp.float32),
            pltpu.VMEM((NTAIL * DIM * N_TICKS,), jnp.float32),
            pltpu.VMEM_SHARED((N_TICKS * S,), jnp.float32),
            pltpu.SemaphoreType.DMA,
            pltpu.SemaphoreType.DMA,
        ],
    )(_body)
    p = k(src32, dst32, t, zt, ztail)
    return bias - (p[0] + p[1])
